# baseline (speedup 1.0000x reference)
import functools

import jax
import jax.numpy as jnp
from jax import lax
from jax.experimental import pallas as pl
from jax.experimental.pallas import tpu as pltpu
from jax.experimental.pallas import tpu_sc as plsc

BATCH = 16384
EMBED_DIM = 32
LANES = 16
NUM_WORKERS = 32
B_PER_W = BATCH // NUM_WORKERS
IDX_CHUNK = 128
GROUPS = B_PER_W // LANES
COL_BLK = 65536


def _matvec_body(w_ref, b_ref, u_ref, o_ref, *, add_bias):
    res = lax.dot_general(w_ref[...], u_ref[...], (((1,), (0,)), ((), ())),
                          preferred_element_type=jnp.float32)
    if add_bias:
        res = res + b_ref[0, 0]
    o_ref[...] = res.reshape(-1)


def _matvec(table_t, w, b2d, add_bias, col_blk):
    n = table_t.shape[1]
    grid = (n + col_blk - 1) // col_blk
    return pl.pallas_call(
        functools.partial(_matvec_body, add_bias=add_bias),
        out_shape=jax.ShapeDtypeStruct((n,), jnp.float32),
        grid=(grid,),
        in_specs=[
            pl.BlockSpec((1, EMBED_DIM), lambda i: (0, 0)),
            pl.BlockSpec((1, 128), lambda i: (0, 0)),
            pl.BlockSpec((EMBED_DIM, col_blk), lambda i: (0, i)),
        ],
        out_specs=pl.BlockSpec((col_blk,), lambda i: (i,)),
    )(w, b2d, table_t)


def _sc_kernel(users_hbm, product_hbm, su_hbm, sp_hbm,
               out_hbm, idx_u, idx_p, suv, spv, out_v, sem):
    nc = 2
    wid = lax.axis_index("s") * nc + lax.axis_index("c")
    base = wid * B_PER_W

    pltpu.sync_copy(users_hbm.at[pl.ds(base, B_PER_W)], idx_u)
    pltpu.sync_copy(product_hbm.at[pl.ds(base, B_PER_W)], idx_p)

    copies = []
    for c in range(B_PER_W // IDX_CHUNK):
        sl = pl.ds(c * IDX_CHUNK, IDX_CHUNK)
        copies.append(pltpu.async_copy(
            su_hbm.at[idx_u.at[sl]], suv.at[sl], sem))
        copies.append(pltpu.async_copy(
            sp_hbm.at[idx_p.at[sl]], spv.at[sl], sem))
    for cp in copies:
        cp.wait()

    def body(g, _):
        sl = pl.ds(g * LANES, LANES)
        out_v[sl] = suv[sl] + spv[sl]
        return ()

    lax.fori_loop(0, GROUPS, body, (), unroll=False)

    pltpu.sync_copy(out_v, out_hbm.at[pl.ds(base, B_PER_W)])


@jax.jit
def _run(users, product, b2d, user_table_t, product_table_t, wu, wp):
    sp = _matvec(product_table_t, wp, b2d, False, COL_BLK)
    su = _matvec(user_table_t, wu, b2d, True, UCOL_BLK)
    mesh = plsc.VectorSubcoreMesh(core_axis_name="c", subcore_axis_name="s")
    f = functools.partial(
        pl.kernel,
        out_type=jax.ShapeDtypeStruct((BATCH,), jnp.float32),
        mesh=mesh,
        compiler_params=pltpu.CompilerParams(
            needs_layout_passes=False, use_tc_tiling_on_sc=False),
        scratch_types=[
            pltpu.VMEM((B_PER_W,), jnp.int32),
            pltpu.VMEM((B_PER_W,), jnp.int32),
            pltpu.VMEM((B_PER_W,), jnp.float32),
            pltpu.VMEM((B_PER_W,), jnp.float32),
            pltpu.VMEM((B_PER_W,), jnp.float32),
            pltpu.SemaphoreType.DMA,
        ],
    )(_sc_kernel)
    return f(users, product, su, sp)


def kernel(users, product, user_table, product_table, W, b):
    b2d = jnp.broadcast_to(b.reshape(1, 1), (1, 128)).astype(jnp.float32)
    out = _run(users.astype(jnp.int32), product.astype(jnp.int32), b2d,
               user_table.T, product_table.T,
               W[:, :EMBED_DIM], W[:, EMBED_DIM:])
    return out.reshape(BATCH, 1)


# speedup vs baseline: 1.0320x; 1.0320x over previous
import functools

import jax
import jax.numpy as jnp
from jax import lax
from jax.experimental import pallas as pl
from jax.experimental.pallas import tpu as pltpu
from jax.experimental.pallas import tpu_sc as plsc

BATCH = 16384
EMBED_DIM = 32
LANES = 16
NUM_WORKERS = 32
B_PER_W = BATCH // NUM_WORKERS
IDX_CHUNK = 128
GROUPS = B_PER_W // LANES
COL_BLK = 65536
UCOL_BLK = 16384
UBLKS = 7


def _matvec_body(w_ref, u_ref, o_ref):
    res = lax.dot_general(w_ref[...], u_ref[...], (((1,), (0,)), ((), ())),
                          preferred_element_type=jnp.float32)
    o_ref[...] = res.reshape(-1)


def _matvec(table_t, w_row):
    n = table_t.shape[1]
    grid = (n + COL_BLK - 1) // COL_BLK
    return pl.pallas_call(
        _matvec_body,
        out_shape=jax.ShapeDtypeStruct((n,), jnp.float32),
        grid=(grid,),
        in_specs=[
            pl.BlockSpec((1, EMBED_DIM), lambda i: (0, 0)),
            pl.BlockSpec((EMBED_DIM, COL_BLK), lambda i: (0, i)),
        ],
        out_specs=pl.BlockSpec((COL_BLK,), lambda i: (i,)),
    )(w_row, table_t)


def _matvec2_body(wu_ref, wp_ref, uu_ref, up_ref, su_ref, sp_ref):
    i = pl.program_id(0)
    nblk_p = pl.num_programs(0) - UBLKS

    @pl.when(i < nblk_p)
    def _():
        res = lax.dot_general(wp_ref[...], up_ref[...],
                              (((1,), (0,)), ((), ())),
                              preferred_element_type=jnp.float32)
        sp_ref[...] = res.reshape(-1)

    @pl.when(i >= nblk_p)
    def _():
        res = lax.dot_general(wu_ref[...], uu_ref[...],
                              (((1,), (0,)), ((), ())),
                              preferred_element_type=jnp.float32)
        su_ref[...] = res.reshape(-1)


def _matvec2(user_t, product_t, wu, wp):
    nu = user_t.shape[1]
    np_ = product_t.shape[1]
    nblk_u = (nu + UCOL_BLK - 1) // UCOL_BLK
    assert nblk_u == UBLKS
    nblk_p = (np_ + COL_BLK - 1) // COL_BLK
    last_p = nblk_p - 1
    return pl.pallas_call(
        _matvec2_body,
        out_shape=(jax.ShapeDtypeStruct((nu,), jnp.float32),
                   jax.ShapeDtypeStruct((np_,), jnp.float32)),
        grid=(nblk_p + nblk_u,),
        in_specs=[
            pl.BlockSpec((1, EMBED_DIM), lambda i: (0, 0)),
            pl.BlockSpec((1, EMBED_DIM), lambda i: (0, 0)),
            pl.BlockSpec((EMBED_DIM, UCOL_BLK),
                         lambda i: (0, jnp.clip(i - (last_p + 1), 0, nblk_u - 1))),
            pl.BlockSpec((EMBED_DIM, COL_BLK),
                         lambda i: (0, jnp.minimum(i, last_p))),
        ],
        out_specs=(
            pl.BlockSpec((UCOL_BLK,),
                         lambda i: (jnp.clip(i - (last_p + 1), 0, nblk_u - 1),)),
            pl.BlockSpec((COL_BLK,), lambda i: (jnp.minimum(i, last_p),)),
        ),
    )(wu, wp, user_t, product_t)


def _sc_kernel(users_hbm, product_hbm, b16_hbm, su_hbm, sp_hbm,
               out_hbm, idx_u, idx_p, suv, spv, bv, out_v, sem):
    nc = 2
    wid = lax.axis_index("s") * nc + lax.axis_index("c")
    base = wid * B_PER_W

    pltpu.sync_copy(users_hbm.at[pl.ds(base, B_PER_W)], idx_u)
    pltpu.sync_copy(product_hbm.at[pl.ds(base, B_PER_W)], idx_p)
    pltpu.sync_copy(b16_hbm, bv)

    copies = []
    for c in range(B_PER_W // IDX_CHUNK):
        sl = pl.ds(c * IDX_CHUNK, IDX_CHUNK)
        copies.append(pltpu.async_copy(
            su_hbm.at[idx_u.at[sl]], suv.at[sl], sem))
        copies.append(pltpu.async_copy(
            sp_hbm.at[idx_p.at[sl]], spv.at[sl], sem))
    for cp in copies:
        cp.wait()

    def body(g, _):
        sl = pl.ds(g * LANES, LANES)
        out_v[sl] = suv[sl] + spv[sl] + bv[...]
        return ()

    lax.fori_loop(0, GROUPS, body, (), unroll=False)

    pltpu.sync_copy(out_v, out_hbm.at[pl.ds(base, B_PER_W)])


@jax.jit
def _run(users, product, b16, user_table_t, product_table_t, wu, wp):
    sp = _matvec(product_table_t, wp)
    su = _matvec(user_table_t, wu)
    su, sp = lax.optimization_barrier((su, sp))
    mesh = plsc.VectorSubcoreMesh(core_axis_name="c", subcore_axis_name="s")
    f = functools.partial(
        pl.kernel,
        out_type=jax.ShapeDtypeStruct((BATCH,), jnp.float32),
        mesh=mesh,
        compiler_params=pltpu.CompilerParams(
            needs_layout_passes=False, use_tc_tiling_on_sc=False),
        scratch_types=[
            pltpu.VMEM((B_PER_W,), jnp.int32),
            pltpu.VMEM((B_PER_W,), jnp.int32),
            pltpu.VMEM((B_PER_W,), jnp.float32),
            pltpu.VMEM((B_PER_W,), jnp.float32),
            pltpu.VMEM((LANES,), jnp.float32),
            pltpu.VMEM((B_PER_W,), jnp.float32),
            pltpu.SemaphoreType.DMA,
        ],
    )(_sc_kernel)
    return f(users, product, b16, su, sp)


def kernel(users, product, user_table, product_table, W, b):
    b16 = jnp.broadcast_to(b, (LANES,)).astype(jnp.float32)
    wu = W[:, :EMBED_DIM]
    wp = W[:, EMBED_DIM:]
    out = _run(users.astype(jnp.int32), product.astype(jnp.int32), b16,
               user_table.T, product_table.T, wu, wp)
    return out.reshape(BATCH, 1)
